# baseline (speedup 1.0000x reference)
import jax
import jax.numpy as jnp
from jax.experimental import pallas as pl

_ROWS = 43008
_D = 512
_T_ELEMS = 672
_BLOCK = 4096


def _copy_kernel(x_ref, o_ref):
    o_ref[...] = x_ref[...]

    @pl.when(pl.program_id(0) == 0)
    def _():
        flat = (jax.lax.broadcasted_iota(jnp.int32, (2, _D), 0) * _D
                + jax.lax.broadcasted_iota(jnp.int32, (2, _D), 1))
        o_ref[0:2, :] = jnp.where(flat < _T_ELEMS, flat.astype(jnp.float32),
                                  x_ref[0:2, :])


def kernel(x):
    xf = x.reshape(_ROWS, _D)
    out = pl.pallas_call(
        _copy_kernel,
        grid=(_ROWS // _BLOCK,),
        in_specs=[pl.BlockSpec((_BLOCK, _D), lambda i: (i, 0))],
        out_specs=pl.BlockSpec((_BLOCK, _D), lambda i: (i, 0)),
        out_shape=jax.ShapeDtypeStruct((_ROWS, _D), jnp.float32),
    )(xf)
    return out.reshape(x.shape)


# speedup vs baseline: 11.0861x; 11.0861x over previous
import jax
import jax.numpy as jnp
from jax.experimental import pallas as pl

_ROWS = 65536
_D = 6 * 7 * 8
_BLOCK = 8192


def _copy_kernel(x_ref, o_ref):
    o_ref[...] = x_ref[...]

    @pl.when(pl.program_id(0) == 0)
    def _():
        flat = (jax.lax.broadcasted_iota(jnp.int32, (2, _D), 0) * _D
                + jax.lax.broadcasted_iota(jnp.int32, (2, _D), 1))
        o_ref[0:2, :] = flat.astype(jnp.float32)


def kernel(x):
    xf = x.reshape(_ROWS, _D)
    out = pl.pallas_call(
        _copy_kernel,
        grid=(_ROWS // _BLOCK,),
        in_specs=[pl.BlockSpec((_BLOCK, _D), lambda i: (i, 0))],
        out_specs=pl.BlockSpec((_BLOCK, _D), lambda i: (i, 0)),
        out_shape=jax.ShapeDtypeStruct((_ROWS, _D), jnp.float32),
    )(xf)
    return out.reshape(x.shape)
